# baseline (speedup 1.0000x reference)
import functools

import jax
import jax.numpy as jnp
from jax import lax
from jax.experimental import pallas as pl
from jax.experimental.pallas import tpu as pltpu
from jax.experimental.pallas import tpu_sc as plsc

_BATCH = 4096
_HIST = 200
_EMB_DIM = 64
_B = _BATCH * _HIST
_NW = 32
_B_PER_W = _B // _NW
_CHUNK = 800
_N_CHUNKS = _B_PER_W // _CHUNK

_mesh = plsc.VectorSubcoreMesh(core_axis_name="c", subcore_axis_name="s")


def _emb_body(x_hbm, table_hbm, out_hbm, idx_v, rows_v, gsem):
    wid = lax.axis_index("s") * 2 + lax.axis_index("c")
    base = wid * _B_PER_W

    def step(g, carry):
        off = base + g * _CHUNK
        pltpu.sync_copy(x_hbm.at[pl.ds(off, _CHUNK)], idx_v)
        pltpu.async_copy(table_hbm.at[idx_v], rows_v, gsem).wait()
        pltpu.sync_copy(rows_v, out_hbm.at[pl.ds(off, _CHUNK)])
        return carry

    lax.fori_loop(0, _N_CHUNKS, step, 0)


_emb = pl.kernel(
    _emb_body,
    out_type=jax.ShapeDtypeStruct((_B, _EMB_DIM), jnp.float32),
    mesh=_mesh,
    scratch_types=[
        pltpu.VMEM((_CHUNK,), jnp.int32),
        pltpu.VMEM((_CHUNK, _EMB_DIM), jnp.float32),
        pltpu.SemaphoreType.DMA,
    ],
    compiler_params=pltpu.CompilerParams(use_tc_tiling_on_sc=False),
)


@jax.jit
def kernel(x, table):
    xf = x.reshape(-1).astype(jnp.int32)
    out = _emb(xf, table)
    return out.reshape(_BATCH, _HIST, _EMB_DIM)


# speedup vs baseline: 1.0261x; 1.0261x over previous
import jax
import jax.numpy as jnp
from jax import lax
from jax.experimental import pallas as pl
from jax.experimental.pallas import tpu as pltpu
from jax.experimental.pallas import tpu_sc as plsc

_BATCH = 4096
_HIST = 200
_EMB_DIM = 64
_B = _BATCH * _HIST
_NW = 32
_B_PER_W = _B // _NW
_CHUNK = 800
_N_CHUNKS = _B_PER_W // _CHUNK

_mesh = plsc.VectorSubcoreMesh(core_axis_name="c", subcore_axis_name="s")


def _emb_body(x_hbm, table_hbm, out_hbm, idx_v, rows0, rows1, gsem0, gsem1,
              osem0, osem1):
    wid = lax.axis_index("s") * 2 + lax.axis_index("c")
    base = wid * _B_PER_W

    pltpu.sync_copy(x_hbm.at[pl.ds(base, _B_PER_W)], idx_v)

    rows = (rows0, rows1)
    gsem = (gsem0, gsem1)
    osem = (osem0, osem1)

    def gather(g, buf):
        return pltpu.make_async_copy(
            table_hbm.at[idx_v.at[pl.ds(g * _CHUNK, _CHUNK)]],
            rows[buf], gsem[buf])

    def flush(g, buf):
        return pltpu.make_async_copy(
            rows[buf], out_hbm.at[pl.ds(base + g * _CHUNK, _CHUNK)],
            osem[buf])

    gather(0, 0).start()

    def step(t, carry):
        g0 = 2 * t
        g1 = g0 + 1

        @pl.when(t > 0)
        def _():
            flush(g0 - 1, 1).wait()

        gather(g1, 1).start()
        gather(g0, 0).wait()
        flush(g0, 0).start()

        flush(g0, 0).wait()

        @pl.when(t < _N_CHUNKS // 2 - 1)
        def _():
            gather(g0 + 2, 0).start()

        gather(g1, 1).wait()
        flush(g1, 1).start()
        return carry

    lax.fori_loop(0, _N_CHUNKS // 2, step, 0)
    flush(_N_CHUNKS - 1, 1).wait()


_emb = pl.kernel(
    _emb_body,
    out_type=jax.ShapeDtypeStruct((_B, _EMB_DIM), jnp.float32),
    mesh=_mesh,
    scratch_types=[
        pltpu.VMEM((_B_PER_W,), jnp.int32),
        pltpu.VMEM((_CHUNK, _EMB_DIM), jnp.float32),
        pltpu.VMEM((_CHUNK, _EMB_DIM), jnp.float32),
        pltpu.SemaphoreType.DMA,
        pltpu.SemaphoreType.DMA,
        pltpu.SemaphoreType.DMA,
        pltpu.SemaphoreType.DMA,
    ],
    compiler_params=pltpu.CompilerParams(use_tc_tiling_on_sc=False),
)


@jax.jit
def kernel(x, table):
    xf = x.reshape(-1).astype(jnp.int32)
    out = _emb(xf, table)
    return out.reshape(_BATCH, _HIST, _EMB_DIM)
